# Initial kernel scaffold; baseline (speedup 1.0000x reference)
import jax
import jax.numpy as jnp
from jax.experimental import pallas as pl


def kernel(index, scores):
    raise NotImplementedError("write your pallas kernel here")


# speedup vs baseline: 12.5464x; 12.5464x over previous
import jax
import jax.numpy as jnp
from jax.experimental import pallas as pl

B, H, L_Q, L_K, N_TOP = 4, 16, 4096, 4096, 64
BH = B * H


def _mask_kernel(idx_ref, out_ref):
    idx = idx_ref[...]
    idx = jnp.swapaxes(idx, 1, 2)
    col = jax.lax.broadcasted_iota(jnp.int32, out_ref.shape, 2)
    out_ref[...] = col > idx


def kernel(index, scores):
    del scores
    ROWS = 8
    idx3 = index.reshape(BH, 1, N_TOP).astype(jnp.int32)
    out = pl.pallas_call(
        _mask_kernel,
        grid=(BH // ROWS,),
        in_specs=[pl.BlockSpec((ROWS, 1, N_TOP), lambda i: (i, 0, 0))],
        out_specs=pl.BlockSpec((ROWS, N_TOP, L_K), lambda i: (i, 0, 0)),
        out_shape=jax.ShapeDtypeStruct((BH, N_TOP, L_K), jnp.bool_),
    )(idx3)
    return out.reshape(B, H, N_TOP, L_K)
